# Initial kernel scaffold; baseline (speedup 1.0000x reference)
import jax
import jax.numpy as jnp
import numpy as np
from jax import lax
from jax.experimental import pallas as pl
from jax.experimental.pallas import tpu as pltpu
from jax.experimental.pallas import tpu_sc as plsc

NC, NS, L = 2, 16, 16
NW = NC * NS
B = 16384
F = 26
PAD_F = 32
ROWS_PER_W = B // NW
SUB = 128
NSUB = ROWS_PER_W // SUB
EIDX_ROWS = SUB * F // 128
SIDX_ROWS = SUB * PAD_F // 128

_OFFS_FM = np.arange(F, dtype=np.int32) * 100000
_OFFS_LIN = np.arange(F, dtype=np.int32) * 1000


def _fm_body(fm_idx_h, fm_sc_h, lin_sc_h, emb_h, wfl_h, wl_h, bias_h,
             out_h, idx_e, idx_f, idx_l, rows_v, fval_v, lval_v, bias_v,
             out_v, sem):
    wid = lax.axis_index("c") * NS + lax.axis_index("s")
    pltpu.sync_copy(bias_h, bias_v)

    @pl.loop(0, NSUB)
    def _(c):
        er0 = wid * (ROWS_PER_W * F // 128) + c * EIDX_ROWS
        sr0 = wid * (ROWS_PER_W * PAD_F // 128) + c * SIDX_ROWS
        pltpu.sync_copy(fm_idx_h.at[pl.ds(er0, EIDX_ROWS)], idx_e)
        pltpu.sync_copy(fm_sc_h.at[pl.ds(sr0, SIDX_ROWS)], idx_f)
        pltpu.sync_copy(lin_sc_h.at[pl.ds(sr0, SIDX_ROWS)], idx_l)

        copies = []
        for j in range(EIDX_ROWS):
            copies.append(pltpu.async_copy(
                emb_h.at[idx_e.at[j]], rows_v.at[pl.ds(j * 128, 128)], sem))
        for j in range(SIDX_ROWS):
            copies.append(pltpu.async_copy(
                wfl_h.at[idx_f.at[j]], fval_v.at[pl.ds(j * 128, 128)], sem))
        for j in range(SIDX_ROWS):
            copies.append(pltpu.async_copy(
                wl_h.at[idx_l.at[j]], lval_v.at[pl.ds(j * 128, 128)], sem))
        for cp in copies:
            cp.wait()

        lane = lax.iota(jnp.int32, 16)
        tail_mask = lane < (F - 16)

        @pl.loop(0, SUB)
        def _(r):
            base = r * F
            s = jnp.zeros((16,), jnp.float32)
            q = jnp.zeros((16,), jnp.float32)
            for f in range(F):
                v = rows_v[base + f]
                s = s + v
                q = q + v * v
            t = s * s - q
            sb = r * PAD_F
            fv1 = fval_v[pl.ds(sb, 16)]
            fv2 = fval_v[pl.ds(sb + 16, 16)]
            lv1 = lval_v[pl.ds(sb, 16)]
            lv2 = lval_v[pl.ds(sb + 16, 16)]
            tail = jnp.where(tail_mask, fv2 + lv2, 0.0)
            z = jnp.sum(0.5 * t + fv1 + lv1 + tail)
            out_v[c * SUB + r] = z

    bias_vec = bias_v[...]

    @pl.loop(0, ROWS_PER_W // 16)
    def _(i):
        zv = out_v[pl.ds(i * 16, 16)] + bias_vec
        out_v[pl.ds(i * 16, 16)] = 1.0 / (1.0 + jnp.exp(-zv))

    pltpu.sync_copy(out_v, out_h.at[pl.ds(wid * ROWS_PER_W, ROWS_PER_W)])


@jax.jit
def _fm_model(fm_idx_g, fm_sc, lin_sc, w_embed, wfl, wl, bias16):
    mesh = plsc.VectorSubcoreMesh(core_axis_name="c", subcore_axis_name="s")
    krn = pl.kernel(
        _fm_body,
        out_type=jax.ShapeDtypeStruct((B,), jnp.float32),
        mesh=mesh,
        scratch_types=[
            pltpu.VMEM((EIDX_ROWS, 128), jnp.int32),
            pltpu.VMEM((SIDX_ROWS, 128), jnp.int32),
            pltpu.VMEM((SIDX_ROWS, 128), jnp.int32),
            pltpu.VMEM((SUB * F, 16), jnp.float32),
            pltpu.VMEM((SUB * PAD_F,), jnp.float32),
            pltpu.VMEM((SUB * PAD_F,), jnp.float32),
            pltpu.VMEM((16,), jnp.float32),
            pltpu.VMEM((ROWS_PER_W,), jnp.float32),
            pltpu.SemaphoreType.DMA,
        ],
    )
    return krn(fm_idx_g, fm_sc, lin_sc, w_embed, wfl, wl, bias16)


def kernel(fm_x, linear_x, W_embed, W_fm_linear, b_fm, W_lin, b_lin):
    fm_idx = fm_x.astype(jnp.int32) + jnp.asarray(_OFFS_FM)[None, :]
    lin_idx = linear_x.astype(jnp.int32) + jnp.asarray(_OFFS_LIN)[None, :]

    fm_idx_g = fm_idx.reshape(B * F // 128, 128)
    pad = jnp.zeros((B, PAD_F - F), jnp.int32)
    fm_sc = jnp.concatenate([fm_idx, pad], axis=1).reshape(B * PAD_F // 128, 128)
    lin_sc = jnp.concatenate([lin_idx, pad], axis=1).reshape(B * PAD_F // 128, 128)

    bias16 = jnp.broadcast_to((b_fm + b_lin).astype(jnp.float32), (16,))
    return _fm_model(fm_idx_g, fm_sc, lin_sc, W_embed,
                     W_fm_linear.reshape(-1), W_lin.reshape(-1), bias16)


# speedup vs baseline: 2.0464x; 2.0464x over previous
import dataclasses

import jax
import jax.numpy as jnp
import numpy as np
from jax import lax
from jax.experimental import pallas as pl
from jax.experimental.pallas import tpu as pltpu
from jax.experimental.pallas import tpu_sc as plsc

NC, NS, L = 2, 16, 16
NW = NC * NS
B = 16384
F = 26
PAD_F = 32
ROWS_PER_W = B // NW
SUB = 128
NSUB = ROWS_PER_W // SUB
EIDX_ROWS = SUB * F // 128
SIDX_ROWS = SUB * PAD_F // 128

_OFFS_FM = np.arange(F, dtype=np.int32) * 100000
_OFFS_LIN = np.arange(F, dtype=np.int32) * 1000


def _fm_body(fm_idx_h, fm_sc_h, lin_sc_h, emb_h, wfl_h, wl_h, bias_h,
             out_h, idx_e, idx_f, idx_l, rows_v, fval_v, lval_v, bias_v,
             out_v, sem):
    wid = lax.axis_index("c") * NS + lax.axis_index("s")
    pltpu.sync_copy(bias_h, bias_v)

    zeros16 = jnp.zeros((16,), jnp.float32)

    @pl.loop(0, ROWS_PER_W // 16)
    def _(i):
        out_v[pl.ds(i * 16, 16)] = zeros16

    pltpu.sync_copy(fm_idx_h.at[pl.ds(wid * (ROWS_PER_W * F // 128),
                                      ROWS_PER_W * F // 128)], idx_e)
    pltpu.sync_copy(fm_sc_h.at[pl.ds(wid * (ROWS_PER_W * PAD_F // 128),
                                     ROWS_PER_W * PAD_F // 128)], idx_f)
    pltpu.sync_copy(lin_sc_h.at[pl.ds(wid * (ROWS_PER_W * PAD_F // 128),
                                      ROWS_PER_W * PAD_F // 128)], idx_l)

    @pl.loop(0, NSUB)
    def _(c):
        copies = []
        for j in range(EIDX_ROWS):
            copies.append(pltpu.async_copy(
                emb_h.at[idx_e.at[c * EIDX_ROWS + j]],
                rows_v.at[pl.ds(j * 128, 128)], sem))
        for j in range(SIDX_ROWS):
            copies.append(pltpu.async_copy(
                wfl_h.at[idx_f.at[c * SIDX_ROWS + j]],
                fval_v.at[pl.ds(j * 128, 128)], sem))
        for j in range(SIDX_ROWS):
            copies.append(pltpu.async_copy(
                wl_h.at[idx_l.at[c * SIDX_ROWS + j]],
                lval_v.at[pl.ds(j * 128, 128)], sem))
        for cp in copies:
            cp.wait()

        lane = lax.iota(jnp.int32, 16)
        tail_mask = lane < (F - 16)

        @pl.loop(0, SUB)
        def _(r):
            base = r * F
            s = jnp.zeros((16,), jnp.float32)
            q = jnp.zeros((16,), jnp.float32)
            for f in range(F):
                v = rows_v[base + f]
                s = s + v
                q = q + v * v
            t = s * s - q
            sb = r * PAD_F
            fv1 = fval_v[pl.ds(sb, 16)]
            fv2 = fval_v[pl.ds(sb + 16, 16)]
            lv1 = lval_v[pl.ds(sb, 16)]
            lv2 = lval_v[pl.ds(sb + 16, 16)]
            tail = jnp.where(tail_mask, fv2 + lv2, 0.0)
            red = 0.5 * t + fv1 + lv1 + tail
            pos = jnp.full((16,), c * SUB + r, jnp.int32)
            plsc.addupdate_scatter(out_v, [pos], red)

    bias_vec = bias_v[...]

    @pl.loop(0, ROWS_PER_W // 16)
    def _(i):
        zv = out_v[pl.ds(i * 16, 16)] + bias_vec
        out_v[pl.ds(i * 16, 16)] = 1.0 / (1.0 + jnp.exp(-zv))

    pltpu.sync_copy(out_v, out_h.at[pl.ds(wid * ROWS_PER_W, ROWS_PER_W)])


@jax.jit
def _fm_model(fm_idx_g, fm_sc, lin_sc, w_embed, wfl, wl, bias16):
    mesh = plsc.VectorSubcoreMesh(core_axis_name="c", subcore_axis_name="s")
    cp = pltpu.CompilerParams()
    for fld, val in (("needs_layout_passes", False),
                     ("use_tc_tiling_on_sc", False)):
        if fld in pltpu.CompilerParams.__dataclass_fields__:
            cp = dataclasses.replace(cp, **{fld: val})
    krn = pl.kernel(
        _fm_body,
        out_type=jax.ShapeDtypeStruct((B,), jnp.float32),
        mesh=mesh,
        compiler_params=cp,
        scratch_types=[
            pltpu.VMEM((ROWS_PER_W * F // 128, 128), jnp.int32),
            pltpu.VMEM((ROWS_PER_W * PAD_F // 128, 128), jnp.int32),
            pltpu.VMEM((ROWS_PER_W * PAD_F // 128, 128), jnp.int32),
            pltpu.VMEM((SUB * F, 16), jnp.float32),
            pltpu.VMEM((SUB * PAD_F,), jnp.float32),
            pltpu.VMEM((SUB * PAD_F,), jnp.float32),
            pltpu.VMEM((16,), jnp.float32),
            pltpu.VMEM((ROWS_PER_W,), jnp.float32),
            pltpu.SemaphoreType.DMA,
        ],
    )
    return krn(fm_idx_g, fm_sc, lin_sc, w_embed, wfl, wl, bias16)


def kernel(fm_x, linear_x, W_embed, W_fm_linear, b_fm, W_lin, b_lin):
    fm_idx = fm_x.astype(jnp.int32) + jnp.asarray(_OFFS_FM)[None, :]
    lin_idx = linear_x.astype(jnp.int32) + jnp.asarray(_OFFS_LIN)[None, :]

    fm_idx_g = fm_idx.reshape(B * F // 128, 128)
    pad = jnp.zeros((B, PAD_F - F), jnp.int32)
    fm_sc = jnp.concatenate([fm_idx, pad], axis=1).reshape(B * PAD_F // 128, 128)
    lin_sc = jnp.concatenate([lin_idx, pad], axis=1).reshape(B * PAD_F // 128, 128)

    bias16 = jnp.broadcast_to((b_fm + b_lin).astype(jnp.float32), (16,))
    return _fm_model(fm_idx_g, fm_sc, lin_sc, W_embed,
                     W_fm_linear.reshape(-1), W_lin.reshape(-1), bias16)


# speedup vs baseline: 3.0457x; 1.4883x over previous
import dataclasses

import jax
import jax.numpy as jnp
import numpy as np
from jax import lax
from jax.experimental import pallas as pl
from jax.experimental.pallas import tpu as pltpu
from jax.experimental.pallas import tpu_sc as plsc

NC, NS, L = 2, 16, 16
NW = NC * NS
B = 16384
F = 26
LIN_V = 26000
ROWS_PER_W = B // NW
SUB = 128
NSUB = ROWS_PER_W // SUB
IDX_ROWS = SUB * F // 128
W_IDX_ROWS = ROWS_PER_W * F // 128
NWIN = SUB * F // 16

_OFFS_FM = np.arange(F, dtype=np.int32) * 100000
_OFFS_LIN = np.arange(F, dtype=np.int32) * 1000
_RID = np.arange(SUB * F, dtype=np.int32) // F


def _fm_body(fm_idx_h, lin_idx_h, emb_h, wfl_h, wl_h, bias_h, rid_h,
             out_h, idx_e, idx_l, rows_v, fval_v, lin_tab, rid_v, bias_v,
             out_v, sem):
    wid = lax.axis_index("c") * NS + lax.axis_index("s")
    pltpu.sync_copy(bias_h, bias_v)
    pltpu.sync_copy(rid_h, rid_v)
    pltpu.sync_copy(wl_h, lin_tab)

    zeros16 = jnp.zeros((16,), jnp.float32)

    @pl.loop(0, ROWS_PER_W // 16)
    def _(i):
        out_v[pl.ds(i * 16, 16)] = zeros16

    pltpu.sync_copy(fm_idx_h.at[pl.ds(wid * W_IDX_ROWS, W_IDX_ROWS)], idx_e)
    pltpu.sync_copy(lin_idx_h.at[pl.ds(wid * ROWS_PER_W * F, ROWS_PER_W * F)],
                    idx_l)

    @pl.loop(0, NSUB)
    def _(c):
        copies = []
        for j in range(IDX_ROWS):
            copies.append(pltpu.async_copy(
                emb_h.at[idx_e.at[c * IDX_ROWS + j]],
                rows_v.at[pl.ds(j * 128, 128)], sem))
        for j in range(IDX_ROWS):
            copies.append(pltpu.async_copy(
                wfl_h.at[idx_e.at[c * IDX_ROWS + j]],
                fval_v.at[pl.ds(j * 128, 128)], sem))
        for cp in copies:
            cp.wait()

        @pl.loop(0, SUB)
        def _(r):
            base = r * F
            s = jnp.zeros((16,), jnp.float32)
            q = jnp.zeros((16,), jnp.float32)
            for f in range(F):
                v = rows_v[base + f]
                s = s + v
                q = q + v * v
            red = 0.5 * (s * s - q)
            pos = jnp.full((16,), c * SUB + r, jnp.int32)
            plsc.addupdate_scatter(out_v, [pos], red)

        @pl.loop(0, NWIN)
        def _(w):
            rows = rid_v[pl.ds(w * 16, 16)] + c * SUB
            fv = fval_v[pl.ds(w * 16, 16)]
            lidx = idx_l[pl.ds(c * SUB * F + w * 16, 16)]
            lv = plsc.load_gather(lin_tab, [lidx])
            plsc.addupdate_scatter(out_v, [rows], fv + lv)

    bias_vec = bias_v[...]

    @pl.loop(0, ROWS_PER_W // 16)
    def _(i):
        zv = out_v[pl.ds(i * 16, 16)] + bias_vec
        out_v[pl.ds(i * 16, 16)] = 1.0 / (1.0 + jnp.exp(-zv))

    pltpu.sync_copy(out_v, out_h.at[pl.ds(wid * ROWS_PER_W, ROWS_PER_W)])


@jax.jit
def _fm_model(fm_idx_g, lin_idx_g, w_embed, wfl, wl, bias16, rid):
    mesh = plsc.VectorSubcoreMesh(core_axis_name="c", subcore_axis_name="s")
    cp = pltpu.CompilerParams()
    for fld, val in (("needs_layout_passes", False),
                     ("use_tc_tiling_on_sc", False)):
        if fld in pltpu.CompilerParams.__dataclass_fields__:
            cp = dataclasses.replace(cp, **{fld: val})
    krn = pl.kernel(
        _fm_body,
        out_type=jax.ShapeDtypeStruct((B,), jnp.float32),
        mesh=mesh,
        compiler_params=cp,
        scratch_types=[
            pltpu.VMEM((W_IDX_ROWS, 128), jnp.int32),
            pltpu.VMEM((W_IDX_ROWS * 128,), jnp.int32),
            pltpu.VMEM((SUB * F, 16), jnp.float32),
            pltpu.VMEM((SUB * F,), jnp.float32),
            pltpu.VMEM((LIN_V,), jnp.float32),
            pltpu.VMEM((SUB * F,), jnp.int32),
            pltpu.VMEM((16,), jnp.float32),
            pltpu.VMEM((ROWS_PER_W,), jnp.float32),
            pltpu.SemaphoreType.DMA,
        ],
    )
    return krn(fm_idx_g, lin_idx_g, w_embed, wfl, wl, bias16, rid)


def kernel(fm_x, linear_x, W_embed, W_fm_linear, b_fm, W_lin, b_lin):
    fm_idx = fm_x.astype(jnp.int32) + jnp.asarray(_OFFS_FM)[None, :]
    lin_idx = linear_x.astype(jnp.int32) + jnp.asarray(_OFFS_LIN)[None, :]

    fm_idx_g = fm_idx.reshape(B * F // 128, 128)
    lin_idx_g = lin_idx.reshape(-1)

    bias16 = jnp.broadcast_to((b_fm + b_lin).astype(jnp.float32), (16,))
    rid = jnp.asarray(_RID)
    return _fm_model(fm_idx_g, lin_idx_g, W_embed,
                     W_fm_linear.reshape(-1), W_lin.reshape(-1), bias16, rid)
